# baseline (speedup 1.0000x reference)
import functools

import jax
import jax.numpy as jnp
from jax import lax
from jax.experimental import pallas as pl
from jax.experimental.pallas import tpu as pltpu
from jax.experimental.pallas import tpu_sc as plsc

N_SEGMENTS = 1_000_000
T_FINAL = 1.0
N_CHANNELS = 16
N_TIMES = 3_276_800

NUM_CORES = 2
NUM_SUBCORES = 16
LANES = 16
NUM_WORKERS = NUM_CORES * NUM_SUBCORES
B_PER_WORKER = N_TIMES // NUM_WORKERS

CHUNK = 2048
N_CHUNKS = B_PER_WORKER // CHUNK
GATHER_W = 128
KG = CHUNK // GATHER_W


def _sc_gather(t_hbm, amp_hbm, out_hbm, t_v, idx_v, rows_v, sem):
    wid = lax.axis_index("s") * NUM_CORES + lax.axis_index("c")
    base = wid * B_PER_WORKER

    def chunk_body(g, carry):
        off = base + g * CHUNK
        pltpu.sync_copy(t_hbm.at[pl.ds(off, CHUNK)], t_v)

        scale = jnp.float32(N_SEGMENTS / T_FINAL)

        def idx_body(r, carry2):
            for c in range(GATHER_W // LANES):
                tv = t_v[pl.ds(r * GATHER_W + c * LANES, LANES)]
                ix = (tv * scale).astype(jnp.int32)
                ix = jnp.minimum(jnp.maximum(ix, 0), N_SEGMENTS - 1)
                idx_v[r, pl.ds(c * LANES, LANES)] = ix
            return carry2

        lax.fori_loop(0, KG, idx_body, 0, unroll=False)

        copies = [
            pltpu.async_copy(
                amp_hbm.at[idx_v.at[j]],
                rows_v.at[pl.ds(j * GATHER_W, GATHER_W)],
                sem,
            )
            for j in range(KG)
        ]
        for cp in copies:
            cp.wait()

        pltpu.sync_copy(rows_v, out_hbm.at[pl.ds(off, CHUNK)])
        return carry

    lax.fori_loop(0, N_CHUNKS, chunk_body, 0, unroll=False)


@jax.jit
def kernel(t, amplitudes):
    mesh = plsc.VectorSubcoreMesh(core_axis_name="c", subcore_axis_name="s")
    run = functools.partial(
        pl.kernel,
        mesh=mesh,
        out_type=jax.ShapeDtypeStruct((N_TIMES, N_CHANNELS), jnp.float32),
        scratch_types=[
            pltpu.VMEM((CHUNK,), jnp.float32),
            pltpu.VMEM((KG, GATHER_W), jnp.int32),
            pltpu.VMEM((CHUNK, N_CHANNELS), jnp.float32),
            pltpu.SemaphoreType.DMA,
        ],
        compiler_params=pltpu.CompilerParams(use_tc_tiling_on_sc=False),
    )(_sc_gather)
    return run(t, amplitudes)


# speedup vs baseline: 1.0425x; 1.0425x over previous
import functools

import jax
import jax.numpy as jnp
from jax import lax
from jax.experimental import pallas as pl
from jax.experimental.pallas import tpu as pltpu
from jax.experimental.pallas import tpu_sc as plsc

N_SEGMENTS = 1_000_000
T_FINAL = 1.0
N_CHANNELS = 16
N_TIMES = 3_276_800

NUM_CORES = 2
NUM_SUBCORES = 16
LANES = 16
NUM_WORKERS = NUM_CORES * NUM_SUBCORES
B_PER_WORKER = N_TIMES // NUM_WORKERS

CHUNK = 2048
N_CHUNKS = B_PER_WORKER // CHUNK
GATHER_W = 128
KG = CHUNK // GATHER_W
SCALE = float(N_SEGMENTS / T_FINAL)


def _sc_gather(t_hbm, amp_hbm, out_hbm,
               t_v0, t_v1, idx_v0, idx_v1, rows0, rows1,
               semt0, semt1, semg0, semg1, semw0, semw1):
    wid = lax.axis_index("s") * NUM_CORES + lax.axis_index("c")
    base = wid * B_PER_WORKER

    def start_t(g, t_v, semt):
        gc = jnp.minimum(g, N_CHUNKS - 1)
        pltpu.async_copy(t_hbm.at[pl.ds(base + gc * CHUNK, CHUNK)], t_v, semt)

    def wait_t(g, t_v, semt):
        gc = jnp.minimum(g, N_CHUNKS - 1)
        pltpu.make_async_copy(
            t_hbm.at[pl.ds(base + gc * CHUNK, CHUNK)], t_v, semt).wait()

    def compute_idx(t_v, idx_v):
        def body(r, carry):
            for c in range(GATHER_W // LANES):
                tv = t_v[pl.ds(r * GATHER_W + c * LANES, LANES)]
                ix = (tv * SCALE).astype(jnp.int32)
                ix = jnp.minimum(jnp.maximum(ix, 0), N_SEGMENTS - 1)
                idx_v[r, pl.ds(c * LANES, LANES)] = ix
            return carry

        lax.fori_loop(0, KG, body, 0, unroll=False)

    def fire_gathers(idx_v, rows_v, semg):
        for j in range(KG):
            pltpu.async_copy(
                amp_hbm.at[idx_v.at[j]],
                rows_v.at[pl.ds(j * GATHER_W, GATHER_W)], semg)

    def drain_gathers(idx_v, rows_v, semg):
        for j in range(KG):
            pltpu.make_async_copy(
                amp_hbm.at[idx_v.at[j]],
                rows_v.at[pl.ds(j * GATHER_W, GATHER_W)], semg).wait()

    def start_wb(g, rows_v, semw):
        pltpu.async_copy(rows_v, out_hbm.at[pl.ds(base + g * CHUNK, CHUNK)],
                         semw)

    def wait_wb(g, rows_v, semw):
        pltpu.make_async_copy(
            rows_v, out_hbm.at[pl.ds(base + g * CHUNK, CHUNK)], semw).wait()

    start_t(0, t_v0, semt0)
    wait_t(0, t_v0, semt0)
    compute_idx(t_v0, idx_v0)
    start_t(1, t_v1, semt1)
    fire_gathers(idx_v0, rows0, semg0)
    wait_t(1, t_v1, semt1)
    compute_idx(t_v1, idx_v1)
    fire_gathers(idx_v1, rows1, semg1)
    start_t(2, t_v0, semt0)
    drain_gathers(idx_v0, rows0, semg0)
    start_wb(0, rows0, semw0)

    def step(s, carry):
        g0 = 2 * s
        g1 = g0 + 1
        wait_t(g0, t_v0, semt0)
        compute_idx(t_v0, idx_v0)
        start_t(g1, t_v1, semt1)
        wait_wb(g0 - 2, rows0, semw0)
        fire_gathers(idx_v0, rows0, semg0)
        drain_gathers(idx_v1, rows1, semg1)
        start_wb(g1 - 2, rows1, semw1)
        wait_t(g1, t_v1, semt1)
        compute_idx(t_v1, idx_v1)
        start_t(g0 + 2, t_v0, semt0)
        wait_wb(g1 - 2, rows1, semw1)
        fire_gathers(idx_v1, rows1, semg1)
        drain_gathers(idx_v0, rows0, semg0)
        start_wb(g0, rows0, semw0)
        return carry

    lax.fori_loop(1, N_CHUNKS // 2, step, 0, unroll=False)

    wait_t(N_CHUNKS, t_v0, semt0)
    drain_gathers(idx_v1, rows1, semg1)
    start_wb(N_CHUNKS - 1, rows1, semw1)
    wait_wb(N_CHUNKS - 2, rows0, semw0)
    wait_wb(N_CHUNKS - 1, rows1, semw1)


@jax.jit
def kernel(t, amplitudes):
    mesh = plsc.VectorSubcoreMesh(core_axis_name="c", subcore_axis_name="s")
    run = functools.partial(
        pl.kernel,
        mesh=mesh,
        out_type=jax.ShapeDtypeStruct((N_TIMES, N_CHANNELS), jnp.float32),
        scratch_types=[
            pltpu.VMEM((CHUNK,), jnp.float32),
            pltpu.VMEM((CHUNK,), jnp.float32),
            pltpu.VMEM((KG, GATHER_W), jnp.int32),
            pltpu.VMEM((KG, GATHER_W), jnp.int32),
            pltpu.VMEM((CHUNK, N_CHANNELS), jnp.float32),
            pltpu.VMEM((CHUNK, N_CHANNELS), jnp.float32),
            pltpu.SemaphoreType.DMA,
            pltpu.SemaphoreType.DMA,
            pltpu.SemaphoreType.DMA,
            pltpu.SemaphoreType.DMA,
            pltpu.SemaphoreType.DMA,
            pltpu.SemaphoreType.DMA,
        ],
        compiler_params=pltpu.CompilerParams(use_tc_tiling_on_sc=False),
    )(_sc_gather)
    return run(t, amplitudes)


# speedup vs baseline: 1.7581x; 1.6864x over previous
import functools

import jax
import jax.numpy as jnp
from jax import lax
from jax.experimental import pallas as pl
from jax.experimental.pallas import tpu as pltpu
from jax.experimental.pallas import tpu_sc as plsc

N_SEGMENTS = 1_000_000
T_FINAL = 1.0
N_CHANNELS = 16
N_TIMES = 3_276_800

NUM_CORES = 2
NUM_SUBCORES = 16
LANES = 16
NUM_WORKERS = NUM_CORES * NUM_SUBCORES
B_PER_WORKER = N_TIMES // NUM_WORKERS

CHUNK = 1024
N_CHUNKS = B_PER_WORKER // CHUNK
GATHER_W = 128
KG = CHUNK // GATHER_W
QB = CHUNK // 128
N_QB = N_TIMES // 128
SCALE = float(N_SEGMENTS / T_FINAL)


def _sc_gather(t_hbm, amp_hbm, out_hbm,
               t_v0, t_v1, idx_v0, idx_v1, rows0, rows1, ob0, ob1,
               semt0, semt1, semg0, semg1, semw0, semw1):
    wid = lax.axis_index("s") * NUM_CORES + lax.axis_index("c")
    base = wid * B_PER_WORKER
    qb_base = wid * (B_PER_WORKER // 128)

    def start_t(g, t_v, semt):
        pltpu.async_copy(t_hbm.at[pl.ds(base + g * CHUNK, CHUNK)], t_v, semt)

    def wait_t(g, t_v, semt):
        pltpu.make_async_copy(
            t_hbm.at[pl.ds(base + g * CHUNK, CHUNK)], t_v, semt).wait()

    def compute_idx(t_v, idx_v):
        def body(r, carry):
            for c in range(GATHER_W // LANES):
                tv = t_v[pl.ds(r * GATHER_W + c * LANES, LANES)]
                ix = (tv * SCALE).astype(jnp.int32)
                ix = jnp.minimum(jnp.maximum(ix, 0), N_SEGMENTS - 1)
                idx_v[r, pl.ds(c * LANES, LANES)] = ix
            return carry

        lax.fori_loop(0, KG, body, 0, unroll=False)

    def fire_gathers(idx_v, rows_v, semg):
        for j in range(KG):
            pltpu.async_copy(
                amp_hbm.at[idx_v.at[j]],
                rows_v.at[pl.ds(j * GATHER_W, GATHER_W)], semg)

    def drain_gathers(idx_v, rows_v, semg):
        for j in range(KG):
            pltpu.make_async_copy(
                amp_hbm.at[idx_v.at[j]],
                rows_v.at[pl.ds(j * GATHER_W, GATHER_W)], semg).wait()

    def transpose(rows_v, obuf):
        iot = lax.iota(jnp.int32, LANES)
        cvecs = [jnp.full((LANES,), ch, jnp.int32) for ch in range(16)]

        def body(k, carry):
            q0 = k * LANES
            qv = iot + q0
            j = q0 // 128
            lane0 = lax.rem(q0, 128)
            for ch in range(16):
                v = plsc.load_gather(rows_v, [qv, cvecs[ch]])
                obuf[ch // 8, j, ch % 8, pl.ds(lane0, LANES)] = v
            return carry

        lax.fori_loop(0, CHUNK // LANES, body, 0, unroll=False)

    def start_wb(g, obuf, semw):
        qb = qb_base + g * QB
        pltpu.async_copy(obuf.at[0], out_hbm.at[0, pl.ds(qb, QB)], semw)
        pltpu.async_copy(obuf.at[1], out_hbm.at[1, pl.ds(qb, QB)], semw)

    def wait_wb(g, obuf, semw):
        qb = qb_base + g * QB
        pltpu.make_async_copy(
            obuf.at[0], out_hbm.at[0, pl.ds(qb, QB)], semw).wait()
        pltpu.make_async_copy(
            obuf.at[1], out_hbm.at[1, pl.ds(qb, QB)], semw).wait()

    slot = [(t_v0, idx_v0, rows0, ob0, semt0, semg0, semw0),
            (t_v1, idx_v1, rows1, ob1, semt1, semg1, semw1)]

    start_t(0, t_v0, semt0)
    start_t(1, t_v1, semt1)
    wait_t(0, t_v0, semt0)
    compute_idx(t_v0, idx_v0)
    start_t(2, t_v0, semt0)
    fire_gathers(idx_v0, rows0, semg0)
    wait_t(1, t_v1, semt1)
    compute_idx(t_v1, idx_v1)
    start_t(3, t_v1, semt1)
    fire_gathers(idx_v1, rows1, semg1)
    drain_gathers(idx_v0, rows0, semg0)
    transpose(rows0, ob0)
    start_wb(0, ob0, semw0)
    wait_t(2, t_v0, semt0)
    compute_idx(t_v0, idx_v0)
    start_t(4, t_v0, semt0)
    fire_gathers(idx_v0, rows0, semg0)
    drain_gathers(idx_v1, rows1, semg1)
    transpose(rows1, ob1)
    start_wb(1, ob1, semw1)
    wait_t(3, t_v1, semt1)
    compute_idx(t_v1, idx_v1)
    start_t(5, t_v1, semt1)
    fire_gathers(idx_v1, rows1, semg1)
    drain_gathers(idx_v0, rows0, semg0)
    wait_wb(0, ob0, semw0)
    transpose(rows0, ob0)
    start_wb(2, ob0, semw0)

    def step(s, carry):
        g0 = 2 * s
        g1 = g0 + 1
        wait_t(g0, t_v0, semt0)
        compute_idx(t_v0, idx_v0)
        start_t(g0 + 2, t_v0, semt0)
        fire_gathers(idx_v0, rows0, semg0)
        drain_gathers(idx_v1, rows1, semg1)
        wait_wb(g0 - 3, ob1, semw1)
        transpose(rows1, ob1)
        start_wb(g0 - 1, ob1, semw1)
        wait_t(g1, t_v1, semt1)
        compute_idx(t_v1, idx_v1)
        start_t(g1 + 2, t_v1, semt1)
        fire_gathers(idx_v1, rows1, semg1)
        drain_gathers(idx_v0, rows0, semg0)
        wait_wb(g1 - 3, ob0, semw0)
        transpose(rows0, ob0)
        start_wb(g0, ob0, semw0)
        return carry

    lax.fori_loop(2, N_CHUNKS // 2 - 1, step, 0, unroll=False)

    gA = N_CHUNKS - 2
    gB = N_CHUNKS - 1
    wait_t(gA, t_v0, semt0)
    compute_idx(t_v0, idx_v0)
    fire_gathers(idx_v0, rows0, semg0)
    drain_gathers(idx_v1, rows1, semg1)
    wait_wb(gA - 3, ob1, semw1)
    transpose(rows1, ob1)
    start_wb(gA - 1, ob1, semw1)

    wait_t(gB, t_v1, semt1)
    compute_idx(t_v1, idx_v1)
    fire_gathers(idx_v1, rows1, semg1)
    drain_gathers(idx_v0, rows0, semg0)
    wait_wb(gB - 3, ob0, semw0)
    transpose(rows0, ob0)
    start_wb(gA, ob0, semw0)

    drain_gathers(idx_v1, rows1, semg1)
    wait_wb(gA - 1, ob1, semw1)
    transpose(rows1, ob1)
    start_wb(gB, ob1, semw1)
    wait_wb(gA, ob0, semw0)
    wait_wb(gB, ob1, semw1)


@jax.jit
def kernel(t, amplitudes):
    mesh = plsc.VectorSubcoreMesh(core_axis_name="c", subcore_axis_name="s")
    run = functools.partial(
        pl.kernel,
        mesh=mesh,
        out_type=jax.ShapeDtypeStruct((2, N_QB, 8, 128), jnp.float32),
        scratch_types=[
            pltpu.VMEM((CHUNK,), jnp.float32),
            pltpu.VMEM((CHUNK,), jnp.float32),
            pltpu.VMEM((KG, GATHER_W), jnp.int32),
            pltpu.VMEM((KG, GATHER_W), jnp.int32),
            pltpu.VMEM((CHUNK, N_CHANNELS), jnp.float32),
            pltpu.VMEM((CHUNK, N_CHANNELS), jnp.float32),
            pltpu.VMEM((2, QB, 8, 128), jnp.float32),
            pltpu.VMEM((2, QB, 8, 128), jnp.float32),
            pltpu.SemaphoreType.DMA,
            pltpu.SemaphoreType.DMA,
            pltpu.SemaphoreType.DMA,
            pltpu.SemaphoreType.DMA,
            pltpu.SemaphoreType.DMA,
            pltpu.SemaphoreType.DMA,
        ],
        compiler_params=pltpu.CompilerParams(
            use_tc_tiling_on_sc=False, needs_layout_passes=False),
    )(_sc_gather)
    out4 = run(t, amplitudes)
    return out4.transpose(1, 3, 0, 2).reshape(N_TIMES, N_CHANNELS)
